# Initial kernel scaffold; baseline (speedup 1.0000x reference)
import jax
import jax.numpy as jnp
from jax.experimental import pallas as pl


def kernel(x, edge_index, idx, Wq0, bq0, Wk0, bk0, Wv0, bv0, Ws0, bs0, Wq1, bq1, Wk1, bk1, Wv1, bv1, Ws1, bs1, W1, b1, W2, b2, W3, b3):
    raise NotImplementedError("write your pallas kernel here")


# speedup vs baseline: 1.0003x; 1.0003x over previous
import jax
import jax.numpy as jnp
from jax.experimental import pallas as pl
from jax.experimental.pallas import tpu as pltpu

N = 10000
E = 320000
D = 128
H = 128
B = 256


def _mlp_body(h_ref, w1_ref, b1_ref, w2_ref, b2_ref, w3_ref, b3_ref, out_ref):
    h = h_ref[...]
    h = jax.nn.relu(jnp.dot(h, w1_ref[...].T, preferred_element_type=jnp.float32) + b1_ref[...])
    h = jax.nn.relu(jnp.dot(h, w2_ref[...].T, preferred_element_type=jnp.float32) + b2_ref[...])
    z = jnp.sum(h * w3_ref[...], axis=-1, keepdims=True) + b3_ref[0]
    out_ref[...] = jax.nn.sigmoid(z)


def _mlp(h, W1, b1, W2, b2, W3, b3):
    return pl.pallas_call(
        _mlp_body,
        out_shape=jax.ShapeDtypeStruct((B, 1), jnp.float32),
    )(h, W1, b1, W2, b2, W3, b3)


def _tconv(x, src, dst, Wq, bq, Wk, bk, Wv, bv, Ws, bs):
    q = x @ Wq.T + bq
    k = x @ Wk.T + bk
    v = x @ Wv.T + bv
    score = jnp.sum(q[dst] * k[src], axis=-1) / jnp.sqrt(jnp.float32(q.shape[-1]))
    smax = jax.ops.segment_max(score, dst, num_segments=N)
    smax = jnp.where(jnp.isfinite(smax), smax, 0.0)
    es = jnp.exp(score - smax[dst])
    ssum = jax.ops.segment_sum(es, dst, num_segments=N)
    alpha = es / (ssum[dst] + 1e-16)
    agg = jax.ops.segment_sum(alpha[:, None] * v[src], dst, num_segments=N)
    return agg + (x @ Ws.T + bs)


def kernel(x, edge_index, idx,
           Wq0, bq0, Wk0, bk0, Wv0, bv0, Ws0, bs0,
           Wq1, bq1, Wk1, bk1, Wv1, bv1, Ws1, bs1,
           W1, b1, W2, b2, W3, b3):
    src = edge_index[0]
    dst = edge_index[1]
    h = _tconv(x, src, dst, Wq0, bq0, Wk0, bk0, Wv0, bv0, Ws0, bs0)
    h = jax.nn.relu(h)
    h = _tconv(h, src, dst, Wq1, bq1, Wk1, bk1, Wv1, bv1, Ws1, bs1)
    h = jax.nn.relu(h)
    h = h[idx]
    return _mlp(h, W1, b1, W2, b2, W3, b3)


# speedup vs baseline: 2.4196x; 2.4189x over previous
import jax
import jax.numpy as jnp
from jax import lax
from jax.experimental import pallas as pl
from jax.experimental.pallas import tpu as pltpu
from jax.experimental.pallas import tpu_sc as plsc

N = 10000
E = 320000
D = 128
H = 128
B = 256

NC = 2
NS = 16
L = 16
NW = NC * NS
EPW = E // NW
G = 80
NCHUNK = EPW // G
AW = H + L
RPS = 624
RTAIL = N - NS * RPS
RC = 16
NRC = N // RC
EPI_K = (NRC + NW - 1) // NW
SPW = B // NW
RSQRT_H = 0.08838834764831845

_MESH = plsc.VectorSubcoreMesh(
    core_axis_name="c", subcore_axis_name="s", num_cores=NC, num_subcores=NS)
_SC_PARAMS = pltpu.CompilerParams(
    needs_layout_passes=False, use_tc_tiling_on_sc=False)


def _wid():
    return lax.axis_index("s") * NC + lax.axis_index("c")


def _proj_body(h_ref, w_ref, b_ref, q_ref, k_ref, v_ref, s_ref):
    acc = jnp.dot(h_ref[...], w_ref[...], preferred_element_type=jnp.float32)
    acc = acc + b_ref[...]
    q_ref[...] = acc[:, :H]
    k_ref[...] = acc[:, H:2 * H]
    v_ref[...] = acc[:, 2 * H:3 * H]
    s_ref[...] = acc[:, 3 * H:]


def _proj(h, wT, b2d):
    bn = 1000
    grid = (N // bn,)
    out = jax.ShapeDtypeStruct((N, H), jnp.float32)
    return pl.pallas_call(
        _proj_body,
        grid=grid,
        in_specs=[
            pl.BlockSpec((bn, D), lambda i: (i, 0)),
            pl.BlockSpec((D, 4 * H), lambda i: (0, 0)),
            pl.BlockSpec((1, 4 * H), lambda i: (0, 0)),
        ],
        out_specs=[pl.BlockSpec((bn, H), lambda i: (i, 0))] * 4,
        out_shape=[out, out, out, out],
    )(h, wT, b2d)


def _mlp_body(h_ref, w1_ref, b1_ref, w2_ref, b2_ref, w3_ref, b3_ref, out_ref):
    h = h_ref[...]
    h = jax.nn.relu(jnp.dot(h, w1_ref[...].T, preferred_element_type=jnp.float32) + b1_ref[...])
    h = jax.nn.relu(jnp.dot(h, w2_ref[...].T, preferred_element_type=jnp.float32) + b2_ref[...])
    z = jnp.sum(h * w3_ref[...], axis=-1, keepdims=True) + b3_ref[0]
    out_ref[...] = jax.nn.sigmoid(z)


def _mlp(h, W1, b1, W2, b2, W3, b3):
    return pl.pallas_call(
        _mlp_body,
        out_shape=jax.ShapeDtypeStruct((B, 1), jnp.float32),
    )(h, W1, b1, W2, b2, W3, b3)


def _score_body(q_hbm, k_hbm, src_hbm, dst_hbm, scores_hbm, pmax_hbm,
                si_v, di_v, qr_v, kr_v, sc_v, mx_v, sem1, sem2):
    wid = _wid()
    ebase = wid * EPW
    mx_v[...] = jnp.full((L,), -3.0e38, jnp.float32)
    lanes = lax.iota(jnp.int32, L)

    def chunk(ci, carry):
        base = pl.multiple_of(ebase + ci * G, 8)
        pltpu.sync_copy(dst_hbm.at[pl.ds(base, G)], di_v)
        pltpu.sync_copy(src_hbm.at[pl.ds(base, G)], si_v)
        cp1 = pltpu.async_copy(q_hbm.at[di_v], qr_v, sem1)
        cp2 = pltpu.async_copy(k_hbm.at[si_v], kr_v, sem2)
        cp1.wait()
        cp2.wait()
        for g in range(G // L):
            evec = jnp.full((L,), g * L, jnp.int32) + lanes

            def jloop(j, acc):
                jvec = jnp.full((L,), j, jnp.int32)
                qg = plsc.load_gather(qr_v, [evec, jvec])
                kg = plsc.load_gather(kr_v, [evec, jvec])
                return acc + qg * kg

            acc = lax.fori_loop(0, H, jloop, jnp.zeros((L,), jnp.float32))
            s = acc * RSQRT_H
            sc_v[pl.ds(g * L, L)] = s
            mx_v[...] = jnp.maximum(mx_v[...], s)
        pltpu.sync_copy(sc_v, scores_hbm.at[pl.ds(base, G)])
        return carry

    lax.fori_loop(0, NCHUNK, chunk, 0)
    pltpu.sync_copy(mx_v, pmax_hbm.at[wid])


def _score(q, k, src, dst):
    f = pl.kernel(
        _score_body,
        out_type=[
            jax.ShapeDtypeStruct((E,), jnp.float32),
            jax.ShapeDtypeStruct((NW, L), jnp.float32),
        ],
        mesh=_MESH,
        compiler_params=_SC_PARAMS,
        scratch_types=[
            pltpu.VMEM((G,), jnp.int32),
            pltpu.VMEM((G,), jnp.int32),
            pltpu.VMEM((G, H), jnp.float32),
            pltpu.VMEM((G, H), jnp.float32),
            pltpu.VMEM((G,), jnp.float32),
            pltpu.VMEM((L,), jnp.float32),
            pltpu.SemaphoreType.DMA,
            pltpu.SemaphoreType.DMA,
        ],
    )
    return f(q, k, src, dst)


def _agg_body(v_hbm, src_hbm, dst_hbm, scores_hbm, pmax_hbm, zeros_hbm,
              accp_hbm,
              si_v, di_v, sc_v, vr_v, vw_v, pm_v, acc_sh, sem1):
    cid = lax.axis_index("c")
    sid = lax.axis_index("s")
    wid = sid * NC + cid
    pltpu.sync_copy(pmax_hbm, pm_v)
    m = pm_v[0, :]
    for r in range(1, NW):
        m = jnp.maximum(m, pm_v[r, :])
    gmax = jnp.max(m)
    rows0 = pl.multiple_of(sid * RPS, 8)
    pltpu.sync_copy(zeros_hbm.at[pl.ds(rows0, RPS)], acc_sh.at[pl.ds(rows0, RPS)])

    @pl.when(sid == NS - 1)
    def _zero_tail():
        t0 = pl.multiple_of(NS * RPS, 8)
        pltpu.sync_copy(zeros_hbm.at[pl.ds(t0, RTAIL)], acc_sh.at[pl.ds(t0, RTAIL)])

    pltpu.sync_copy(zeros_hbm.at[pl.ds(0, G)], vw_v)
    plsc.subcore_barrier()
    ebase = wid * EPW
    lanes = lax.iota(jnp.int32, L)
    colH = jnp.full((L,), H, jnp.int32)

    def chunk(ci, carry):
        base = pl.multiple_of(ebase + ci * G, 8)
        pltpu.sync_copy(dst_hbm.at[pl.ds(base, G)], di_v)
        pltpu.sync_copy(src_hbm.at[pl.ds(base, G)], si_v)
        pltpu.sync_copy(scores_hbm.at[pl.ds(base, G)], sc_v)
        pltpu.async_copy(v_hbm.at[si_v], vr_v, sem1).wait()
        for g in range(G // L):
            es = jnp.exp(sc_v[pl.ds(g * L, L)] - gmax)
            evec = jnp.full((L,), g * L, jnp.int32) + lanes

            def jloop(j, c):
                jvec = jnp.full((L,), j, jnp.int32)
                vg = plsc.load_gather(vr_v, [evec, jvec])
                plsc.store_scatter(vw_v, [evec, jvec], vg * es)
                return c

            lax.fori_loop(0, H, jloop, 0)
            plsc.store_scatter(vw_v, [evec, colH], es)
        pltpu.sync_copy(vw_v, acc_sh.at[di_v], add=True)
        return carry

    lax.fori_loop(0, NCHUNK, chunk, 0)
    plsc.subcore_barrier()
    pltpu.sync_copy(acc_sh.at[pl.ds(rows0, RPS)],
                    accp_hbm.at[cid, pl.ds(rows0, RPS)])

    @pl.when(sid == NS - 1)
    def _wb_tail():
        t0 = pl.multiple_of(NS * RPS, 8)
        pltpu.sync_copy(acc_sh.at[pl.ds(t0, RTAIL)],
                        accp_hbm.at[cid, pl.ds(t0, RTAIL)])


def _agg(v, src, dst, scores, pmax, zeros):
    f = pl.kernel(
        _agg_body,
        out_type=jax.ShapeDtypeStruct((NC, N, AW), jnp.float32),
        mesh=_MESH,
        compiler_params=_SC_PARAMS,
        scratch_types=[
            pltpu.VMEM((G,), jnp.int32),
            pltpu.VMEM((G,), jnp.int32),
            pltpu.VMEM((G,), jnp.float32),
            pltpu.VMEM((G, H), jnp.float32),
            pltpu.VMEM((G, AW), jnp.float32),
            pltpu.VMEM((NW, L), jnp.float32),
            pltpu.VMEM_SHARED((N, AW), jnp.float32),
            pltpu.SemaphoreType.DMA,
        ],
    )
    return f(v, src, dst, scores, pmax, zeros)


def _epi_body(accp_hbm, s_hbm, h_hbm, a0_v, a1_v, sv_v, hv_v):
    wid = _wid()
    lanes = lax.iota(jnp.int32, L)
    colH = jnp.full((L,), H, jnp.int32)

    def step(k, carry):
        t = wid + k * NW

        @pl.when(t < NRC)
        def _():
            r0 = pl.multiple_of(t * RC, 8)
            pltpu.sync_copy(accp_hbm.at[0, pl.ds(r0, RC)], a0_v)
            pltpu.sync_copy(accp_hbm.at[1, pl.ds(r0, RC)], a1_v)
            pltpu.sync_copy(s_hbm.at[pl.ds(r0, RC)], sv_v)
            d0 = plsc.load_gather(a0_v, [lanes, colH])
            d1 = plsc.load_gather(a1_v, [lanes, colH])
            rden = 1.0 / (d0 + d1 + 1e-16)

            def cloop(c, cc):
                cvec = jnp.full((L,), c, jnp.int32)
                num = (plsc.load_gather(a0_v, [lanes, cvec])
                       + plsc.load_gather(a1_v, [lanes, cvec]))
                sk = plsc.load_gather(sv_v, [lanes, cvec])
                hcol = jnp.maximum(sk + num * rden, 0.0)
                plsc.store_scatter(hv_v, [lanes, cvec], hcol)
                return cc

            lax.fori_loop(0, H, cloop, 0)
            pltpu.sync_copy(hv_v, h_hbm.at[pl.ds(r0, RC)])

        return carry

    lax.fori_loop(0, EPI_K, step, 0)


def _epi(accp, s):
    f = pl.kernel(
        _epi_body,
        out_type=jax.ShapeDtypeStruct((N, H), jnp.float32),
        mesh=_MESH,
        compiler_params=_SC_PARAMS,
        scratch_types=[
            pltpu.VMEM((RC, AW), jnp.float32),
            pltpu.VMEM((RC, AW), jnp.float32),
            pltpu.VMEM((RC, H), jnp.float32),
            pltpu.VMEM((RC, H), jnp.float32),
        ],
    )
    return f(accp, s)


def _sel_body(h_hbm, idx_hbm, out_hbm, idx_v, rows_v, sem):
    wid = _wid()
    base = pl.multiple_of(wid * SPW, 8)
    pltpu.sync_copy(idx_hbm.at[pl.ds(base, SPW)], idx_v)
    pltpu.async_copy(h_hbm.at[idx_v], rows_v, sem).wait()
    pltpu.sync_copy(rows_v, out_hbm.at[pl.ds(base, SPW)])


def _sel(h, idx):
    f = pl.kernel(
        _sel_body,
        out_type=jax.ShapeDtypeStruct((B, H), jnp.float32),
        mesh=_MESH,
        compiler_params=_SC_PARAMS,
        scratch_types=[
            pltpu.VMEM((SPW,), jnp.int32),
            pltpu.VMEM((SPW, H), jnp.float32),
            pltpu.SemaphoreType.DMA,
        ],
    )
    return f(h, idx)


def _layer(h, src, dst, wT, b2d, zeros):
    q, k, v, s = _proj(h, wT, b2d)
    scores, pmax = _score(q, k, src, dst)
    accp = _agg(v, src, dst, scores, pmax, zeros)
    return _epi(accp, s)


def kernel(x, edge_index, idx,
           Wq0, bq0, Wk0, bk0, Wv0, bv0, Ws0, bs0,
           Wq1, bq1, Wk1, bk1, Wv1, bv1, Ws1, bs1,
           W1, b1, W2, b2, W3, b3):
    src = edge_index[0]
    dst = edge_index[1]
    zeros = jnp.zeros((N, AW), jnp.float32)
    w0T = jnp.concatenate([Wq0, Wk0, Wv0, Ws0], axis=0).T
    b0 = jnp.concatenate([bq0, bk0, bv0, bs0]).reshape(1, 4 * H)
    w1T = jnp.concatenate([Wq1, Wk1, Wv1, Ws1], axis=0).T
    b1c = jnp.concatenate([bq1, bk1, bv1, bs1]).reshape(1, 4 * H)
    h = _layer(x, src, dst, w0T, b0, zeros)
    h = _layer(h, src, dst, w1T, b1c, zeros)
    hsel = _sel(h, idx)
    return _mlp(hsel, W1, b1, W2, b2, W3, b3)
